# baseline (speedup 1.0000x reference)
import jax
import jax.numpy as jnp
from jax.experimental import pallas as pl
from jax.experimental.pallas import tpu as pltpu

_COMMIT_BETA = 0.25
_ROWS = 2048


def _split_bf16(x):
    hi = x.astype(jnp.bfloat16)
    lo = (x - hi.astype(jnp.float32)).astype(jnp.bfloat16)
    return hi, lo


def _mm(a, b, dims):
    return jax.lax.dot_general(a, b, (dims, ((), ())),
                               preferred_element_type=jnp.float32)


def _vq_block(z_ref, cb_ref, enc_ref, idx_ref, loss_ref,
              cnorm_ref, ch_ref, cl_ref, clo32_ref, lacc_ref, macc_ref):
    i = pl.program_id(0)
    nblk = pl.num_programs(0)
    z = z_ref[...]
    K = cb_ref.shape[0]
    R = z.shape[0]

    @pl.when(i == 0)
    def _():
        cb = cb_ref[...]
        ones_col = jnp.ones((cb.shape[1], 1), jnp.float32)
        cnorm_ref[...] = jax.lax.dot_general(
            cb * cb, ones_col, (((1,), (0,)), ((), ())),
            precision=jax.lax.Precision.HIGHEST,
            preferred_element_type=jnp.float32)
        m2h, m2l = _split_bf16(-2.0 * cb)
        ch_ref[...] = m2h
        cl_ref[...] = m2l
        clo32_ref[...] = cb - cb.astype(jnp.bfloat16).astype(jnp.float32)
        lacc_ref[...] = jnp.zeros_like(lacc_ref)
        macc_ref[...] = jnp.zeros_like(macc_ref)

    c_norm = cnorm_ref[...]
    ch = ch_ref[...]
    cl = cl_ref[...]
    zh, zl = _split_bf16(z)
    cdims = ((1,), (1,))
    dotsT = (_mm(ch, zh, cdims) + _mm(ch, zl, cdims) + _mm(cl, zh, cdims))
    scoresT = c_norm + dotsT
    min_val = jnp.min(scoresT, axis=0)
    kiota = jax.lax.broadcasted_iota(jnp.int32, (K, R), 0)
    big = jnp.int32(2**30)
    idx = jnp.min(jnp.where(scoresT == min_val[None, :], kiota, big), axis=0)
    onehotT = (kiota == idx[None, :]).astype(jnp.float32)
    gdims = ((0,), (0,))
    enc = _mm(onehotT, cb_ref[...], gdims) + _mm(onehotT, clo32_ref[...], gdims)
    enc_ref[...] = enc
    idx_ref[...] = idx.reshape(idx_ref.shape)
    lacc_ref[...] += jnp.sum(z * z, axis=0, keepdims=True)
    macc_ref[...] += min_val[None, :]

    @pl.when(i == nblk - 1)
    def _():
        loss_ref[...] = (jnp.sum(lacc_ref[...]) +
                         jnp.sum(macc_ref[...])).reshape(1, 1)


def kernel(z, codebook):
    B, w, h, c = z.shape
    K = codebook.shape[0]
    N = B * w * h
    zf = z.reshape(N, c)
    grid = N // _ROWS
    enc, idx, loss = pl.pallas_call(
        _vq_block,
        grid=(grid,),
        in_specs=[
            pl.BlockSpec((_ROWS, c), lambda i: (i, 0)),
            pl.BlockSpec((K, c), lambda i: (0, 0)),
        ],
        out_specs=[
            pl.BlockSpec((_ROWS, c), lambda i: (i, 0)),
            pl.BlockSpec((1, 1, _ROWS), lambda i: (i, 0, 0)),
            pl.BlockSpec((1, 1), lambda i: (0, 0)),
        ],
        out_shape=[
            jax.ShapeDtypeStruct((N, c), jnp.float32),
            jax.ShapeDtypeStruct((grid, 1, _ROWS), jnp.int32),
            jax.ShapeDtypeStruct((1, 1), jnp.float32),
        ],
        scratch_shapes=[
            pltpu.VMEM((K, 1), jnp.float32),
            pltpu.VMEM((K, c), jnp.bfloat16),
            pltpu.VMEM((K, c), jnp.bfloat16),
            pltpu.VMEM((K, c), jnp.float32),
            pltpu.VMEM((1, c), jnp.float32),
            pltpu.VMEM((1, _ROWS), jnp.float32),
        ],
    )(zf, codebook)
    quantized = enc.reshape(B, w, h, c)
    embedding_indexes = idx.reshape(B, w, h)
    embedding_loss = loss[0, 0] / jnp.float32(N * c)
    commitment_loss = _COMMIT_BETA * embedding_loss
    return quantized, embedding_indexes, embedding_loss, commitment_loss


# speedup vs baseline: 1.0413x; 1.0413x over previous
import jax
import jax.numpy as jnp
from jax.experimental import pallas as pl
from jax.experimental.pallas import tpu as pltpu

_COMMIT_BETA = 0.25
_ROWS = 2048


def _split_bf16(x):
    hi = x.astype(jnp.bfloat16)
    lo = (x - hi.astype(jnp.float32)).astype(jnp.bfloat16)
    return hi, lo


def _mm(a, b, dims):
    return jax.lax.dot_general(a, b, (dims, ((), ())),
                               preferred_element_type=jnp.float32)


def _vq_block(z_ref, cb_ref, enc_ref, idx_ref, loss_ref,
              cnorm_ref, ch_ref, cl_ref, clo32_ref, lacc_ref):
    i = pl.program_id(0)
    nblk = pl.num_programs(0)
    z = z_ref[...]
    K = cb_ref.shape[0]
    R = z.shape[0]

    @pl.when(i == 0)
    def _():
        cb = cb_ref[...]
        ones_col = jnp.ones((cb.shape[1], 1), jnp.float32)
        cnorm_ref[...] = jax.lax.dot_general(
            cb * cb, ones_col, (((1,), (0,)), ((), ())),
            precision=jax.lax.Precision.HIGHEST,
            preferred_element_type=jnp.float32)
        hi, lo = _split_bf16(cb)
        ch_ref[...] = hi
        cl_ref[...] = lo
        clo32_ref[...] = cb - cb.astype(jnp.bfloat16).astype(jnp.float32)
        lacc_ref[...] = jnp.zeros_like(lacc_ref)

    c_norm = cnorm_ref[...]
    ch = ch_ref[...]
    cl = cl_ref[...]
    zh, zl = _split_bf16(z)
    cdims = ((1,), (1,))
    dotsT = (_mm(ch, zh, cdims) + _mm(ch, zl, cdims) + _mm(cl, zh, cdims))
    scoresT = c_norm - 2.0 * dotsT
    min_val = jnp.min(scoresT, axis=0)
    kiota = jax.lax.broadcasted_iota(jnp.int32, (K, R), 0)
    big = jnp.int32(2**30)
    idx = jnp.min(jnp.where(scoresT == min_val[None, :], kiota, big), axis=0)
    onehotT = (kiota == idx[None, :]).astype(jnp.float32)
    gdims = ((0,), (0,))
    enc = _mm(onehotT, cb_ref[...], gdims) + _mm(onehotT, clo32_ref[...], gdims)
    enc_ref[...] = enc
    idx_ref[...] = idx.reshape(idx_ref.shape)
    lacc_ref[...] += jnp.sum((z - enc) ** 2, axis=0, keepdims=True)

    @pl.when(i == nblk - 1)
    def _():
        loss_ref[...] = jnp.sum(lacc_ref[...]).reshape(1, 1)


def kernel(z, codebook):
    B, w, h, c = z.shape
    K = codebook.shape[0]
    N = B * w * h
    zf = z.reshape(N, c)
    grid = N // _ROWS
    enc, idx, loss = pl.pallas_call(
        _vq_block,
        grid=(grid,),
        in_specs=[
            pl.BlockSpec((_ROWS, c), lambda i: (i, 0)),
            pl.BlockSpec((K, c), lambda i: (0, 0)),
        ],
        out_specs=[
            pl.BlockSpec((_ROWS, c), lambda i: (i, 0)),
            pl.BlockSpec((1, 1, _ROWS), lambda i: (i, 0, 0)),
            pl.BlockSpec((1, 1), lambda i: (0, 0)),
        ],
        out_shape=[
            jax.ShapeDtypeStruct((N, c), jnp.float32),
            jax.ShapeDtypeStruct((grid, 1, _ROWS), jnp.int32),
            jax.ShapeDtypeStruct((1, 1), jnp.float32),
        ],
        scratch_shapes=[
            pltpu.VMEM((K, 1), jnp.float32),
            pltpu.VMEM((K, c), jnp.bfloat16),
            pltpu.VMEM((K, c), jnp.bfloat16),
            pltpu.VMEM((K, c), jnp.float32),
            pltpu.VMEM((1, c), jnp.float32),
        ],
    )(zf, codebook)
    quantized = enc.reshape(B, w, h, c)
    embedding_indexes = idx.reshape(B, w, h)
    embedding_loss = loss[0, 0] / jnp.float32(N * c)
    commitment_loss = _COMMIT_BETA * embedding_loss
    return quantized, embedding_indexes, embedding_loss, commitment_loss


# speedup vs baseline: 1.0487x; 1.0072x over previous
import jax
import jax.numpy as jnp
from jax.experimental import pallas as pl
from jax.experimental.pallas import tpu as pltpu

_COMMIT_BETA = 0.25
_ROWS = 2048


def _split_bf16(x):
    hi = x.astype(jnp.bfloat16)
    lo = (x - hi.astype(jnp.float32)).astype(jnp.bfloat16)
    return hi, lo


def _mm(a, b, dims):
    return jax.lax.dot_general(a, b, (dims, ((), ())),
                               preferred_element_type=jnp.float32)


def _vq_block(z_ref, cb_ref, enc_ref, idx_ref, loss_ref,
              cnorm_ref, ch_ref, cl_ref, clo32_ref, lacc_ref):
    i = pl.program_id(0)
    nblk = pl.num_programs(0)
    z = z_ref[...]
    K = cb_ref.shape[0]
    R = z.shape[0]

    @pl.when(i == 0)
    def _():
        cb = cb_ref[...]
        ones_col = jnp.ones((cb.shape[1], 1), jnp.float32)
        cnorm_ref[...] = jax.lax.dot_general(
            cb * cb, ones_col, (((1,), (0,)), ((), ())),
            precision=jax.lax.Precision.HIGHEST,
            preferred_element_type=jnp.float32)
        hi, lo = _split_bf16(cb)
        ch_ref[...] = hi
        cl_ref[...] = lo
        clo32_ref[...] = cb - cb.astype(jnp.bfloat16).astype(jnp.float32)
        lacc_ref[...] = jnp.zeros_like(lacc_ref)

    c_norm = cnorm_ref[...]
    ch = ch_ref[...]
    cl = cl_ref[...]
    zh, zl = _split_bf16(z)
    cdims = ((1,), (1,))
    dotsT = (_mm(ch, zh, cdims) + _mm(ch, zl, cdims) + _mm(cl, zh, cdims))
    scoresT = c_norm - 2.0 * dotsT
    min_val = jnp.min(scoresT, axis=0)
    kiota = jax.lax.broadcasted_iota(jnp.int32, (K, R), 0)
    big = jnp.int32(2**30)
    idx = jnp.min(jnp.where(scoresT == min_val[None, :], kiota, big), axis=0)
    onehotT = (kiota == idx[None, :]).astype(jnp.bfloat16)
    gdims = ((0,), (0,))
    enc = _mm(onehotT, ch, gdims) + _mm(onehotT, cl, gdims)
    enc_ref[...] = enc
    idx_ref[...] = idx.reshape(idx_ref.shape)
    lacc_ref[...] += jnp.sum((z - enc) ** 2, axis=0, keepdims=True)

    @pl.when(i == nblk - 1)
    def _():
        loss_ref[...] = jnp.sum(lacc_ref[...]).reshape(1, 1)


def kernel(z, codebook):
    B, w, h, c = z.shape
    K = codebook.shape[0]
    N = B * w * h
    zf = z.reshape(N, c)
    grid = N // _ROWS
    enc, idx, loss = pl.pallas_call(
        _vq_block,
        grid=(grid,),
        in_specs=[
            pl.BlockSpec((_ROWS, c), lambda i: (i, 0)),
            pl.BlockSpec((K, c), lambda i: (0, 0)),
        ],
        out_specs=[
            pl.BlockSpec((_ROWS, c), lambda i: (i, 0)),
            pl.BlockSpec((1, 1, _ROWS), lambda i: (i, 0, 0)),
            pl.BlockSpec((1, 1), lambda i: (0, 0)),
        ],
        out_shape=[
            jax.ShapeDtypeStruct((N, c), jnp.float32),
            jax.ShapeDtypeStruct((grid, 1, _ROWS), jnp.int32),
            jax.ShapeDtypeStruct((1, 1), jnp.float32),
        ],
        scratch_shapes=[
            pltpu.VMEM((K, 1), jnp.float32),
            pltpu.VMEM((K, c), jnp.bfloat16),
            pltpu.VMEM((K, c), jnp.bfloat16),
            pltpu.VMEM((K, c), jnp.float32),
            pltpu.VMEM((1, c), jnp.float32),
        ],
    )(zf, codebook)
    quantized = enc.reshape(B, w, h, c)
    embedding_indexes = idx.reshape(B, w, h)
    embedding_loss = loss[0, 0] / jnp.float32(N * c)
    commitment_loss = _COMMIT_BETA * embedding_loss
    return quantized, embedding_indexes, embedding_loss, commitment_loss
